# baseline (speedup 1.0000x reference)
import functools

import jax
import jax.numpy as jnp
from jax import lax
from jax.experimental import pallas as pl
from jax.experimental.pallas import tpu as pltpu
from jax.experimental.pallas import tpu_sc as plsc


@functools.lru_cache(maxsize=None)
def _make_gather(num_workers, rows_per_worker, D):
    num_cores = 2
    mesh = plsc.VectorSubcoreMesh(core_axis_name="c", subcore_axis_name="s")

    @functools.partial(
        pl.kernel,
        mesh=mesh,
        out_type=jax.ShapeDtypeStruct((num_workers, rows_per_worker, D), jnp.float32),
        scratch_types=[
            pltpu.VMEM((rows_per_worker,), jnp.int32),
            pltpu.VMEM((rows_per_worker, D), jnp.float32),
            pltpu.SemaphoreType.DMA,
        ],
    )
    def gather_kernel(idx_hbm, table_hbm, out_hbm, idx_v, rows_v, sem):
        wid = lax.axis_index("s") * num_cores + lax.axis_index("c")
        pltpu.sync_copy(idx_hbm.at[wid], idx_v)
        pltpu.async_copy(table_hbm.at[idx_v], rows_v, sem).wait()
        pltpu.sync_copy(rows_v, out_hbm.at[wid])

    return gather_kernel


def kernel(indices, table):
    D = table.shape[1]
    idx_flat = indices.reshape(-1).astype(jnp.int32)
    B = idx_flat.shape[0]
    num_workers = 32
    rows_per_worker = B // num_workers
    idx3 = idx_flat.reshape(num_workers, rows_per_worker)
    out = _make_gather(num_workers, rows_per_worker, D)(idx3, table)
    return out.reshape(indices.shape + (D,))


# speedup vs baseline: 1.1513x; 1.1513x over previous
import functools

import jax
import jax.numpy as jnp
from jax import lax
from jax.experimental import pallas as pl
from jax.experimental.pallas import tpu as pltpu
from jax.experimental.pallas import tpu_sc as plsc


@functools.lru_cache(maxsize=None)
def _make_gather(B, D, rows_per_worker, n_chunks):
    num_workers = B // rows_per_worker
    chunk = rows_per_worker // n_chunks
    mesh = plsc.VectorSubcoreMesh(
        core_axis_name="c", subcore_axis_name="s", num_cores=1
    )

    @functools.partial(
        pl.kernel,
        mesh=mesh,
        out_type=jax.ShapeDtypeStruct((B, D), jnp.float32),
        scratch_types=[
            pltpu.VMEM((rows_per_worker,), jnp.int32),
            pltpu.VMEM((n_chunks, chunk, D), jnp.float32),
            pltpu.SemaphoreType.DMA((n_chunks,)),
        ],
    )
    def gather_kernel(idx_hbm, table_hbm, out_hbm, idx_v, rows_v, sems):
        wid = lax.axis_index("s")

        @pl.when(wid < num_workers)
        def _():
            base = wid * rows_per_worker
            pltpu.sync_copy(idx_hbm.at[pl.ds(base, rows_per_worker)], idx_v)
            copies = []
            for j in range(n_chunks):
                copies.append(
                    pltpu.async_copy(
                        table_hbm.at[idx_v.at[pl.ds(j * chunk, chunk)]],
                        rows_v.at[j],
                        sems.at[j],
                    )
                )
            for j in range(n_chunks):
                copies[j].wait()
                pltpu.sync_copy(
                    rows_v.at[j], out_hbm.at[pl.ds(base + j * chunk, chunk)]
                )

    return gather_kernel


def kernel(indices, table):
    D = table.shape[1]
    idx_flat = indices.reshape(-1).astype(jnp.int32)
    B = idx_flat.shape[0]
    out = _make_gather(B, D, 24, 3)(idx_flat, table)
    return out.reshape(indices.shape + (D,))
